# Initial kernel scaffold; baseline (speedup 1.0000x reference)
import jax
import jax.numpy as jnp
from jax.experimental import pallas as pl


def kernel(x, edge_index, edge_weight, W, b, gamma, beta):
    raise NotImplementedError("write your pallas kernel here")


# speedup vs baseline: 15.2658x; 15.2658x over previous
import functools

import jax
import jax.numpy as jnp
from jax import lax
from jax.experimental import pallas as pl
from jax.experimental.pallas import tpu as pltpu
from jax.experimental.pallas import tpu_sc as plsc

N = 10000
NPAD = 10240
D = 128
E = 320000

NC = 2
NS = 16
NW = NC * NS
CHUNK = 128
NCHUNK = 81
EPW = NCHUNK * CHUNK
EPAD = NW * EPW
RPT = NPAD // NS
EPS = 1e-5


def _dis_body(deg_ref, o_ref):
    d = deg_ref[0:1, :] + deg_ref[1:2, :]
    o_ref[...] = lax.rsqrt(jnp.maximum(d, 1e-30))


def _k2b_dis(degs):
    return pl.pallas_call(
        _dis_body,
        out_shape=jax.ShapeDtypeStruct((1, NPAD), jnp.float32),
    )(degs)


def _mm_body(x_ref, w_ref, o_ref):
    o_ref[...] = jnp.dot(x_ref[...], w_ref[...],
                         preferred_element_type=jnp.float32)


def _k1_matmul(x_pad, W):
    return pl.pallas_call(
        _mm_body,
        grid=(NPAD // 512,),
        in_specs=[
            pl.BlockSpec((512, D), lambda i: (i, 0)),
            pl.BlockSpec((D, D), lambda i: (0, 0)),
        ],
        out_specs=pl.BlockSpec((512, D), lambda i: (i, 0)),
        out_shape=jax.ShapeDtypeStruct((NPAD, D), jnp.float32),
    )(x_pad, W)


def _k2_body(dst_hbm, w_hbm, deg_out, idx_v, w_v, zbuf, deg_sp, sem):
    c = lax.axis_index("c")
    s = lax.axis_index("s")
    wid = c * NS + s

    z = jnp.zeros((16,), jnp.float32)
    def zb(i, _):
        zbuf[pl.ds(i * 16, 16)] = z
        return 0
    lax.fori_loop(0, RPT // 16, zb, 0)
    pltpu.sync_copy(zbuf, deg_sp.at[pl.ds(s * RPT, RPT)])
    plsc.subcore_barrier()

    def chunk(k, _):
        base = wid * EPW + k * CHUNK
        pltpu.sync_copy(dst_hbm.at[pl.ds(base, CHUNK)], idx_v)
        pltpu.sync_copy(w_hbm.at[pl.ds(base, CHUNK)], w_v)
        pltpu.sync_copy(w_v, deg_sp.at[idx_v], add=True)
        return 0
    lax.fori_loop(0, NCHUNK, chunk, 0)
    plsc.subcore_barrier()

    pltpu.sync_copy(deg_sp.at[pl.ds(s * RPT, RPT)], zbuf)
    pltpu.sync_copy(zbuf, deg_out.at[c, pl.ds(s * RPT, RPT)])


def _k2_degree(dst_f, w_f):
    mesh = plsc.VectorSubcoreMesh(core_axis_name="c", subcore_axis_name="s")
    f = functools.partial(
        pl.kernel,
        out_type=jax.ShapeDtypeStruct((NC, NPAD), jnp.float32),
        mesh=mesh,
        compiler_params=pltpu.CompilerParams(needs_layout_passes=False),
        scratch_types=[
            pltpu.VMEM((CHUNK,), jnp.int32),
            pltpu.VMEM((CHUNK,), jnp.float32),
            pltpu.VMEM((RPT,), jnp.float32),
            pltpu.VMEM_SHARED((NPAD,), jnp.float32),
            pltpu.SemaphoreType.DMA,
        ],
    )(_k2_body)
    return f(dst_f, w_f)


def _k3_body(src_hbm, dst_hbm, w_hbm, dis_hbm, xw_hbm, part_out,
             dis_v, sidx_v, didx_v, w_v, norm_v, rows_v, acc_sp, sem):
    c = lax.axis_index("c")
    s = lax.axis_index("s")
    wid = c * NS + s

    pltpu.sync_copy(dis_hbm, dis_v)

    z = jnp.zeros((16,), jnp.float32)
    def zrow(i, _):
        r = i // 8
        f = i % 8
        rows_v[r, pl.ds(f * 16, 16)] = z
        return 0
    lax.fori_loop(0, CHUNK * 8, zrow, 0)
    for k in range(RPT // CHUNK):
        pltpu.sync_copy(rows_v, acc_sp.at[pl.ds(s * RPT + k * CHUNK, CHUNK)])
    plsc.subcore_barrier()

    def chunk(k, _):
        base = wid * EPW + k * CHUNK
        pltpu.sync_copy(src_hbm.at[pl.ds(base, CHUNK)], sidx_v)
        pltpu.sync_copy(dst_hbm.at[pl.ds(base, CHUNK)], didx_v)
        pltpu.sync_copy(w_hbm.at[pl.ds(base, CHUNK)], w_v)
        gat = pltpu.async_copy(xw_hbm.at[sidx_v], rows_v, sem)
        def nrm(i, _):
            si = sidx_v[pl.ds(i * 16, 16)]
            di = didx_v[pl.ds(i * 16, 16)]
            ds_ = plsc.load_gather(dis_v, [si])
            dd_ = plsc.load_gather(dis_v, [di])
            norm_v[pl.ds(i * 16, 16)] = ds_ * w_v[pl.ds(i * 16, 16)] * dd_
            return 0
        lax.fori_loop(0, CHUNK // 16, nrm, 0)
        gat.wait()
        def scale(j, _):
            nv = norm_v[pl.ds(j * 16, 16)]
            for lane in range(16):
                e = j * 16 + lane
                sc = nv[lane]
                for f in range(8):
                    rows_v[e, pl.ds(f * 16, 16)] = (
                        rows_v[e, pl.ds(f * 16, 16)] * sc)
            return 0
        lax.fori_loop(0, CHUNK // 16, scale, 0)
        pltpu.sync_copy(rows_v, acc_sp.at[didx_v], add=True)
        return 0
    lax.fori_loop(0, NCHUNK, chunk, 0)
    plsc.subcore_barrier()

    for k in range(RPT // CHUNK):
        row = s * RPT + k * CHUNK
        pltpu.sync_copy(acc_sp.at[pl.ds(row, CHUNK)], rows_v)
        pltpu.sync_copy(rows_v, part_out.at[c, pl.ds(row, CHUNK)])


def _k3_aggregate(src_f, dst_f, w_f, dis, xw):
    mesh = plsc.VectorSubcoreMesh(core_axis_name="c", subcore_axis_name="s")
    f = functools.partial(
        pl.kernel,
        out_type=jax.ShapeDtypeStruct((NC, NPAD, D), jnp.float32),
        mesh=mesh,
        compiler_params=pltpu.CompilerParams(needs_layout_passes=False),
        scratch_types=[
            pltpu.VMEM((NPAD,), jnp.float32),
            pltpu.VMEM((CHUNK,), jnp.int32),
            pltpu.VMEM((CHUNK,), jnp.int32),
            pltpu.VMEM((CHUNK,), jnp.float32),
            pltpu.VMEM((CHUNK,), jnp.float32),
            pltpu.VMEM((CHUNK, D), jnp.float32),
            pltpu.VMEM_SHARED((NPAD, D), jnp.float32),
            pltpu.SemaphoreType.DMA,
        ],
    )(_k3_body)
    return f(src_f, dst_f, w_f, dis, xw)


def _stats_body(p0_ref, p1_ref, b_ref, o_ref):
    i = pl.program_id(0)
    v = p0_ref[...] + p1_ref[...] + b_ref[...]
    rows = i * 512 + lax.broadcasted_iota(jnp.int32, (512, D), 0)
    v = jnp.where(rows < N, v, 0.0)
    blk = jnp.stack([jnp.sum(v, 0), jnp.sum(v * v, 0)])

    @pl.when(i == 0)
    def _():
        o_ref[...] = blk

    @pl.when(i > 0)
    def _():
        o_ref[...] = o_ref[...] + blk


def _k4a_stats(p0, p1, b2):
    return pl.pallas_call(
        _stats_body,
        grid=(NPAD // 512,),
        in_specs=[
            pl.BlockSpec((512, D), lambda i: (i, 0)),
            pl.BlockSpec((512, D), lambda i: (i, 0)),
            pl.BlockSpec((1, D), lambda i: (0, 0)),
        ],
        out_specs=pl.BlockSpec((2, D), lambda i: (0, 0)),
        out_shape=jax.ShapeDtypeStruct((2, D), jnp.float32),
    )(p0, p1, b2)


def _bn_body(p0_ref, p1_ref, b_ref, st_ref, g_ref, bt_ref, o_ref):
    v = p0_ref[...] + p1_ref[...] + b_ref[...]
    mean = st_ref[0:1, :] / N
    var = st_ref[1:2, :] / N - mean * mean
    inv = lax.rsqrt(var + EPS)
    h = (v - mean) * inv * g_ref[...] + bt_ref[...]
    o_ref[...] = jnp.maximum(h, 0.0)


def _k4b_bn(p0, p1, b2, stats, g2, bt2):
    return pl.pallas_call(
        _bn_body,
        grid=(N // 1000,),
        in_specs=[
            pl.BlockSpec((1000, D), lambda i: (i, 0)),
            pl.BlockSpec((1000, D), lambda i: (i, 0)),
            pl.BlockSpec((1, D), lambda i: (0, 0)),
            pl.BlockSpec((2, D), lambda i: (0, 0)),
            pl.BlockSpec((1, D), lambda i: (0, 0)),
            pl.BlockSpec((1, D), lambda i: (0, 0)),
        ],
        out_specs=pl.BlockSpec((1000, D), lambda i: (i, 0)),
        out_shape=jax.ShapeDtypeStruct((N, D), jnp.float32),
    )(p0, p1, b2, stats, g2, bt2)


def kernel(x, edge_index, edge_weight, W, b, gamma, beta):
    src = edge_index[0]
    dst = edge_index[1]
    loop = jnp.arange(N, dtype=jnp.int32)
    npad_e = EPAD - E - N
    pad_idx = jnp.arange(npad_e, dtype=jnp.int32)
    src_f = jnp.concatenate([src, loop, pad_idx])
    dst_f = jnp.concatenate([dst, loop, pad_idx])
    w_f = jnp.concatenate([edge_weight, jnp.ones((N,), jnp.float32),
                           jnp.zeros((npad_e,), jnp.float32)])
    x_pad = jnp.pad(x, ((0, NPAD - N), (0, 0)))

    xw = _k1_matmul(x_pad, W)
    degs = _k2_degree(dst_f, w_f)
    dis = _k2b_dis(degs).reshape(NPAD)
    parts = _k3_aggregate(src_f, dst_f, w_f, dis, xw)

    b2 = b.reshape(1, D)
    g2 = gamma.reshape(1, D)
    bt2 = beta.reshape(1, D)
    stats = _k4a_stats(parts[0], parts[1], b2)
    return _k4b_bn(parts[0], parts[1], b2, stats, g2, bt2)
